# Initial kernel scaffold; baseline (speedup 1.0000x reference)
import jax
import jax.numpy as jnp
from jax.experimental import pallas as pl


def kernel(R_bb, t_bb, alpha, aa, restype_rigid_group_rotation, restype_rigid_group_translation, restype_atom14_to_rigid_group, restype_atom14_rigid_group_positions):
    raise NotImplementedError("write your pallas kernel here")


# speedup vs baseline: 92.9810x; 92.9810x over previous
import functools

import jax
import jax.numpy as jnp
from jax import lax
from jax.experimental import pallas as pl
from jax.experimental.pallas import tpu as pltpu
from jax.experimental.pallas import tpu_sc as plsc

_LANES = 16
_SCR = 256


def _splat(v):
    return jnp.full((_LANES,), v, dtype=jnp.int32)


def _rsqrt(x):
    i = plsc.bitcast(x, jnp.int32)
    i = jnp.int32(0x5F3759DF) - (i >> 1)
    y = plsc.bitcast(i, jnp.float32)
    for _ in range(3):
        y = y * (1.5 - 0.5 * x * y * y)
    return y


@functools.lru_cache(maxsize=None)
def _build(tot):
    info = plsc.get_sparse_core_info()
    nc, ns = info.num_cores, info.num_subcores
    nw = nc * ns
    per_w = tot // nw
    assert per_w % _SCR == 0 and tot == per_w * nw

    mesh = plsc.VectorSubcoreMesh(core_axis_name="c", subcore_axis_name="s")

    @functools.partial(
        pl.kernel,
        mesh=mesh,
        out_type=jax.ShapeDtypeStruct((tot, 42), jnp.float32),
        compiler_params=pltpu.CompilerParams(
            needs_layout_passes=False, use_tc_tiling_on_sc=False),
        scratch_types=[
            pltpu.VMEM((21, 72), jnp.float32),
            pltpu.VMEM((21, 24), jnp.float32),
            pltpu.VMEM((21, 14), jnp.int32),
            pltpu.VMEM((21, 42), jnp.float32),
            pltpu.VMEM((_SCR, 9), jnp.float32),
            pltpu.VMEM((_SCR, 3), jnp.float32),
            pltpu.VMEM((_SCR, 10), jnp.float32),
            pltpu.VMEM((_SCR,), jnp.int32),
            pltpu.VMEM((_SCR, 42), jnp.float32),
            pltpu.VMEM((6, 12, 128), jnp.float32),
        ],
    )
    def sc_kernel(rbb_h, tbb_h, al_h, aa_h, rot_h, trn_h, grp_h, pos_h,
                  out_h,
                  rot_v, trn_v, grp_v, pos_v, rbb_v, tbb_v, al_v, aa_v,
                  out_v, frm_v):
        pltpu.sync_copy(rot_h, rot_v)
        pltpu.sync_copy(trn_h, trn_v)
        pltpu.sync_copy(grp_h, grp_v)
        pltpu.sync_copy(pos_h, pos_v)

        wid = lax.axis_index("s") * nc + lax.axis_index("c")
        base_w = wid * per_w
        lane = lax.iota(jnp.int32, _LANES)

        def chunk(c, carry):
            res = lane + c * _LANES

            def gin(ref, col):
                return plsc.load_gather(ref, [res, _splat(col)])

            aa16 = aa_v[pl.ds(c * _LANES, _LANES)]

            def gtab(ref, col):
                return plsc.load_gather(ref, [aa16, _splat(col)])

            R = [[gin(rbb_v, 3 * i + j) for j in range(3)] for i in range(3)]
            t = [gin(tbb_v, j) for j in range(3)]

            sincos = []
            for k in range(5):
                a0 = gin(al_v, 2 * k)
                a1 = gin(al_v, 2 * k + 1)
                inv = _rsqrt(a0 * a0 + a1 * a1 + 1e-8)
                sincos.append((a0 * inv, a1 * inv))

            def compose_frame(Rp, tp, f, sin, cos):
                rf = [[gtab(rot_v, f * 9 + 3 * i + j) for j in range(3)]
                      for i in range(3)]
                tf = [gtab(trn_v, f * 3 + j) for j in range(3)]
                Ra = [[Rp[i][0] * rf[0][j] + Rp[i][1] * rf[1][j]
                       + Rp[i][2] * rf[2][j] for j in range(3)]
                      for i in range(3)]
                ta = [Rp[i][0] * tf[0] + Rp[i][1] * tf[1]
                      + Rp[i][2] * tf[2] + tp[i] for i in range(3)]
                Rb = [[Ra[i][0],
                       cos * Ra[i][1] + sin * Ra[i][2],
                       cos * Ra[i][2] - sin * Ra[i][1]] for i in range(3)]
                return Rb, ta

            frames = [(R, t)]
            frames.append(compose_frame(R, t, 3, *sincos[0]))
            Rc, tc = R, t
            for k in range(4):
                Rc, tc = compose_frame(Rc, tc, 4 + k, *sincos[1 + k])
                frames.append((Rc, tc))

            for f, (Rf, tf) in enumerate(frames):
                for i in range(3):
                    for j in range(3):
                        plsc.store_scatter(
                            frm_v, [_splat(f), _splat(3 * i + j), lane],
                            Rf[i][j])
                    plsc.store_scatter(
                        frm_v, [_splat(f), _splat(9 + i), lane], tf[i])

            for a in range(14):
                g = gtab(grp_v, a)
                m = jnp.maximum(g - 2, 0)
                p = [gtab(pos_v, 3 * a + w) for w in range(3)]
                Rm = [plsc.load_gather(frm_v, [m, _splat(w), lane])
                      for w in range(9)]
                tm = [plsc.load_gather(frm_v, [m, _splat(9 + w), lane])
                      for w in range(3)]
                for i in range(3):
                    o = (Rm[3 * i] * p[0] + Rm[3 * i + 1] * p[1]
                         + Rm[3 * i + 2] * p[2] + tm[i])
                    plsc.store_scatter(out_v, [res, _splat(3 * a + i)], o)
            return carry

        def superchunk(s, carry):
            base = base_w + s * _SCR
            pltpu.sync_copy(rbb_h.at[pl.ds(base, _SCR)], rbb_v)
            pltpu.sync_copy(tbb_h.at[pl.ds(base, _SCR)], tbb_v)
            pltpu.sync_copy(al_h.at[pl.ds(base, _SCR)], al_v)
            pltpu.sync_copy(aa_h.at[pl.ds(base, _SCR)], aa_v)
            lax.fori_loop(0, _SCR // _LANES, chunk, 0)
            pltpu.sync_copy(out_v, out_h.at[pl.ds(base, _SCR)])
            return carry

        lax.fori_loop(0, per_w // _SCR, superchunk, 0)

    return sc_kernel


def kernel(R_bb, t_bb, alpha, aa,
           restype_rigid_group_rotation,
           restype_rigid_group_translation,
           restype_atom14_to_rigid_group,
           restype_atom14_rigid_group_positions):
    N, L = aa.shape
    tot = N * L
    out = _build(tot)(
        R_bb.reshape(tot, 9),
        t_bb.reshape(tot, 3),
        alpha.reshape(tot, 10),
        aa.reshape(tot).astype(jnp.int32),
        restype_rigid_group_rotation.reshape(21, 72),
        restype_rigid_group_translation.reshape(21, 24),
        restype_atom14_to_rigid_group.astype(jnp.int32),
        restype_atom14_rigid_group_positions.reshape(21, 42),
    )
    return out.reshape(N, L, 14, 3)
